# Initial kernel scaffold; baseline (speedup 1.0000x reference)
import functools

import jax
import jax.numpy as jnp
from jax import lax
from jax.experimental import pallas as pl
from jax.experimental.pallas import tpu as pltpu
from jax.experimental.pallas import tpu_sc as plsc

N_POINTS = 100000
CIN = 64
COUT = 64
NK = 16
BEV_H = 256
BEV_W = 256
BATCH = 4
NROWS = BATCH * BEV_H * BEV_W

NC = 2
NS = 16
NW = NC * NS

P_TILE = 3456
N_PAD = NW * P_TILE
BLK = 1152
SUB = 128
NSUB = BLK // SUB
NBLK = P_TILE // BLK

CHUNK = 26624
NCHUNKS = 10
NCPC = NCHUNKS // NC
ROWS_PER_TILE = CHUNK // NS
ZROWS = 416

MM_BLK = 1024
MM_GRID = N_PAD // MM_BLK


def _matmul_tc(featp, kidxp, x0p, x1p, bp, kern2):
    def body(feat_ref, kidx_ref, x0_ref, x1_ref, b_ref, kern_ref,
             sf_ref, flat_ref):
        feat = feat_ref[...]
        kidx = kidx_ref[...]
        parts = [
            jnp.where(kidx == k, feat, 0.0).astype(jnp.bfloat16)
            for k in range(NK)
        ]
        big = jnp.concatenate(parts, axis=1)
        kern = kern_ref[...].astype(jnp.bfloat16)
        sf_ref[...] = jnp.dot(big, kern, preferred_element_type=jnp.float32)
        flat_ref[...] = (b_ref[...] * (BEV_H * BEV_W)
                         + x0_ref[...] * BEV_W + x1_ref[...])

    int_spec = pl.BlockSpec((MM_BLK, 1), lambda i: (i, 0))
    sf, flat2 = pl.pallas_call(
        body,
        grid=(MM_GRID,),
        in_specs=[
            pl.BlockSpec((MM_BLK, CIN), lambda i: (i, 0)),
            int_spec, int_spec, int_spec, int_spec,
            pl.BlockSpec((NK * CIN, COUT), lambda i: (0, 0)),
        ],
        out_specs=[
            pl.BlockSpec((MM_BLK, COUT), lambda i: (i, 0)),
            pl.BlockSpec((MM_BLK, 1), lambda i: (i, 0)),
        ],
        out_shape=[
            jax.ShapeDtypeStruct((N_PAD, COUT), jnp.float32),
            jax.ShapeDtypeStruct((N_PAD, 1), jnp.int32),
        ],
    )(featp, kidxp, x0p, x1p, bp, kern2)
    return sf, flat2.reshape(N_PAD)


def _scatter_sc(rows, flat, zeros_hbm):
    mesh = plsc.VectorSubcoreMesh(
        core_axis_name="c", subcore_axis_name="s",
        num_cores=NC, num_subcores=NS)

    lidx_types = [pltpu.VMEM((SUB,), jnp.int32) for _ in range(NSUB)]

    @functools.partial(
        pl.kernel,
        out_type=jax.ShapeDtypeStruct((NCHUNKS * CHUNK, COUT), jnp.float32),
        mesh=mesh,
        scratch_types=[
            pltpu.VMEM((P_TILE,), jnp.int32),
            pltpu.VMEM((BLK, COUT), jnp.float32),
            pltpu.VMEM((ZROWS, COUT), jnp.float32),
            pltpu.VMEM_SHARED((CHUNK + 8, COUT), jnp.float32),
        ] + lidx_types,
    )
    def k(rows_hbm, flat_hbm, zsrc_hbm, bev_hbm, idx_v, rows_v, zbuf, acc,
          *lidx):
        cid = lax.axis_index("c")
        sid = lax.axis_index("s")
        wid = sid * NC + cid
        base_pt = wid * P_TILE

        pltpu.sync_copy(flat_hbm.at[pl.ds(base_pt, P_TILE)], idx_v)
        pltpu.sync_copy(zsrc_hbm, zbuf)

        def per_chunk(p, carry):
            chunk_id = cid * NCPC + p
            chunk_base = chunk_id * CHUNK

            for z in range(ROWS_PER_TILE // ZROWS):
                pltpu.sync_copy(
                    zbuf,
                    acc.at[pl.ds(sid * ROWS_PER_TILE + z * ZROWS, ZROWS), :])
            plsc.subcore_barrier()

            for bkt in range(NBLK):
                pltpu.sync_copy(
                    rows_hbm.at[pl.ds(base_pt + bkt * BLK, BLK), :], rows_v)
                for j in range(NSUB):
                    def sub_body(i2, c, _j=j, _bkt=bkt):
                        off = _bkt * BLK + _j * SUB + i2 * 16
                        v = idx_v[pl.ds(off, 16)]
                        loc = v - chunk_base
                        ok = (loc >= 0) & (loc < CHUNK)
                        loc = jnp.where(ok, loc, CHUNK)
                        lidx[_j][pl.ds(i2 * 16, 16)] = loc
                        return c
                    lax.fori_loop(0, SUB // 16, sub_body, 0)
                    pltpu.sync_copy(
                        rows_v.at[pl.ds(j * SUB, SUB), :],
                        acc.at[lidx[j]],
                        add=True)
            plsc.subcore_barrier()

            pltpu.sync_copy(
                acc.at[pl.ds(sid * ROWS_PER_TILE, ROWS_PER_TILE), :],
                bev_hbm.at[pl.ds(chunk_base + sid * ROWS_PER_TILE,
                                 ROWS_PER_TILE), :])
            plsc.subcore_barrier()
            return carry

        lax.fori_loop(0, NCPC, per_chunk, 0)

    return k(rows, flat, zeros_hbm)


def _transpose_tc(bev):
    HW = BEV_H * BEV_W
    TB = 4096

    def body(in_ref, out_ref):
        out_ref[...] = in_ref[...].T[None]

    return pl.pallas_call(
        body,
        grid=(BATCH, HW // TB),
        in_specs=[pl.BlockSpec((TB, COUT),
                               lambda b, i: (b * (HW // TB) + i, 0))],
        out_specs=pl.BlockSpec((1, COUT, TB), lambda b, i: (b, 0, i)),
        out_shape=jax.ShapeDtypeStruct((BATCH, COUT, HW), jnp.float32),
    )(bev)


def kernel(features, coords_d0, coords_d1, coords_d2, coords_b, stride,
           kernel):
    kidx = (coords_d1 // stride).astype(jnp.int32)
    x0 = (coords_d0 // stride).astype(jnp.int32)
    x1 = (coords_d2 // stride).astype(jnp.int32)
    b = coords_b.astype(jnp.int32)

    pad = N_PAD - N_POINTS
    featp = jnp.concatenate([features, jnp.zeros((pad, CIN), jnp.float32)])
    zi = jnp.zeros((pad,), jnp.int32)
    kidxp = jnp.concatenate([kidx, zi]).reshape(N_PAD, 1)
    x0p = jnp.concatenate([x0, zi]).reshape(N_PAD, 1)
    x1p = jnp.concatenate([x1, zi]).reshape(N_PAD, 1)
    bp = jnp.concatenate([b, zi - 1]).reshape(N_PAD, 1)
    kern2 = kernel.reshape(NK * CIN, COUT)

    sf, flat = _matmul_tc(featp, kidxp, x0p, x1p, bp, kern2)

    zeros_hbm = jnp.zeros((ZROWS, COUT), jnp.float32)
    bev_pad = _scatter_sc(sf, flat, zeros_hbm)
    bev = bev_pad[:NROWS]

    out = _transpose_tc(bev)
    return out.reshape(BATCH, COUT, BEV_H, BEV_W)


# speedup vs baseline: 1.6058x; 1.6058x over previous
import functools

import jax
import jax.numpy as jnp
from jax import lax
from jax.experimental import pallas as pl
from jax.experimental.pallas import tpu as pltpu
from jax.experimental.pallas import tpu_sc as plsc

N_POINTS = 100000
CIN = 64
COUT = 64
NK = 16
BEV_H = 256
BEV_W = 256
BATCH = 4
NROWS = BATCH * BEV_H * BEV_W

NC = 2
NS = 16
NW = NC * NS

N_PAD = 114688
P_TILE = N_PAD // NS
BLK = 512
NBLK = P_TILE // BLK

CHUNK = 21888
NCHUNKS = 12
NCPC = NCHUNKS // NC
ROWS_PER_TILE = CHUNK // NS

MM_BLK = 1024
MM_GRID = N_PAD // MM_BLK


def _matmul_tc(featp, kidxp, x0p, x1p, bp, kern2):
    def body(feat_ref, kidx_ref, x0_ref, x1_ref, b_ref, kern_ref,
             sf_ref, flat_ref):
        feat = feat_ref[...]
        kidx = kidx_ref[...]
        parts = [
            jnp.where(kidx == k, feat, 0.0).astype(jnp.bfloat16)
            for k in range(NK)
        ]
        big = jnp.concatenate(parts, axis=1)
        kern = kern_ref[...].astype(jnp.bfloat16)
        sf_ref[...] = jnp.dot(big, kern, preferred_element_type=jnp.float32)
        flat_ref[...] = (b_ref[...] * (BEV_H * BEV_W)
                         + x0_ref[...] * BEV_W + x1_ref[...])

    int_spec = pl.BlockSpec((MM_BLK, 1), lambda i: (i, 0))
    sf, flat2 = pl.pallas_call(
        body,
        grid=(MM_GRID,),
        in_specs=[
            pl.BlockSpec((MM_BLK, CIN), lambda i: (i, 0)),
            int_spec, int_spec, int_spec, int_spec,
            pl.BlockSpec((NK * CIN, COUT), lambda i: (0, 0)),
        ],
        out_specs=[
            pl.BlockSpec((MM_BLK, COUT), lambda i: (i, 0)),
            pl.BlockSpec((MM_BLK, 1), lambda i: (i, 0)),
        ],
        out_shape=[
            jax.ShapeDtypeStruct((N_PAD, COUT), jnp.float32),
            jax.ShapeDtypeStruct((N_PAD, 1), jnp.int32),
        ],
    )(featp, kidxp, x0p, x1p, bp, kern2)
    return sf, flat2.reshape(N_PAD)


def _scatter_sc(rows, flat, zeros_hbm):
    mesh = plsc.VectorSubcoreMesh(
        core_axis_name="c", subcore_axis_name="s",
        num_cores=NC, num_subcores=NS)

    @functools.partial(
        pl.kernel,
        out_type=jax.ShapeDtypeStruct((NCHUNKS * CHUNK, COUT), jnp.float32),
        mesh=mesh,
        compiler_params=pltpu.CompilerParams(use_tc_tiling_on_sc=False),
        scratch_types=[
            pltpu.VMEM((P_TILE,), jnp.int32),
            pltpu.VMEM((BLK, COUT), jnp.float32),
            pltpu.VMEM((BLK,), jnp.int32),
            pltpu.VMEM_SHARED((CHUNK + 8, COUT), jnp.float32),
        ],
    )
    def k(rows_hbm, flat_hbm, zsrc_hbm, bev_hbm, idx_v, rows_v, lidx, acc):
        cid = lax.axis_index("c")
        sid = lax.axis_index("s")
        base_pt = sid * P_TILE

        pltpu.sync_copy(flat_hbm.at[pl.ds(base_pt, P_TILE)], idx_v)

        def per_chunk(p, carry):
            chunk_id = cid * NCPC + p
            chunk_base = chunk_id * CHUNK

            pltpu.sync_copy(
                zsrc_hbm,
                acc.at[pl.ds(sid * ROWS_PER_TILE, ROWS_PER_TILE), :])
            plsc.subcore_barrier()

            for bkt in range(NBLK):
                pltpu.sync_copy(
                    rows_hbm.at[pl.ds(base_pt + bkt * BLK, BLK), :], rows_v)

                def sub_body(i2, c, _bkt=bkt):
                    off = _bkt * BLK + i2 * 16
                    v = idx_v[pl.ds(off, 16)]
                    loc = v - chunk_base
                    ok = (loc >= 0) & (loc < CHUNK)
                    loc = jnp.where(ok, loc, CHUNK)
                    lidx[pl.ds(i2 * 16, 16)] = loc
                    return c
                lax.fori_loop(0, BLK // 16, sub_body, 0)
                pltpu.sync_copy(rows_v, acc.at[lidx], add=True)
            plsc.subcore_barrier()

            pltpu.sync_copy(
                acc.at[pl.ds(sid * ROWS_PER_TILE, ROWS_PER_TILE), :],
                bev_hbm.at[pl.ds(chunk_base + sid * ROWS_PER_TILE,
                                 ROWS_PER_TILE), :])
            plsc.subcore_barrier()
            return carry

        lax.fori_loop(0, NCPC, per_chunk, 0)

    return k(rows, flat, zeros_hbm)


def _transpose_tc(bev):
    HW = BEV_H * BEV_W
    TB = 4096

    def body(in_ref, out_ref):
        out_ref[...] = in_ref[...].T[None]

    return pl.pallas_call(
        body,
        grid=(BATCH, HW // TB),
        in_specs=[pl.BlockSpec((TB, COUT),
                               lambda b, i: (b * (HW // TB) + i, 0))],
        out_specs=pl.BlockSpec((1, COUT, TB), lambda b, i: (b, 0, i)),
        out_shape=jax.ShapeDtypeStruct((BATCH, COUT, HW), jnp.float32),
    )(bev)


def kernel(features, coords_d0, coords_d1, coords_d2, coords_b, stride,
           kernel):
    kidx = (coords_d1 // stride).astype(jnp.int32)
    x0 = (coords_d0 // stride).astype(jnp.int32)
    x1 = (coords_d2 // stride).astype(jnp.int32)
    b = coords_b.astype(jnp.int32)

    pad = N_PAD - N_POINTS
    featp = jnp.concatenate([features, jnp.zeros((pad, CIN), jnp.float32)])
    zi = jnp.zeros((pad,), jnp.int32)
    kidxp = jnp.concatenate([kidx, zi]).reshape(N_PAD, 1)
    x0p = jnp.concatenate([x0, zi]).reshape(N_PAD, 1)
    x1p = jnp.concatenate([x1, zi]).reshape(N_PAD, 1)
    bp = jnp.concatenate([b, zi - 1]).reshape(N_PAD, 1)
    kern2 = kernel.reshape(NK * CIN, COUT)

    sf, flat = _matmul_tc(featp, kidxp, x0p, x1p, bp, kern2)

    zeros_hbm = jnp.zeros((ROWS_PER_TILE, COUT), jnp.float32)
    bev_pad = _scatter_sc(sf, flat, zeros_hbm)
    bev = bev_pad[:NROWS]

    out = _transpose_tc(bev)
    return out.reshape(BATCH, COUT, BEV_H, BEV_W)


# speedup vs baseline: 2.2942x; 1.4287x over previous
import functools

import jax
import jax.numpy as jnp
from jax import lax
from jax.experimental import pallas as pl
from jax.experimental.pallas import tpu as pltpu
from jax.experimental.pallas import tpu_sc as plsc

N_POINTS = 100000
CIN = 64
COUT = 64
NK = 16
BEV_H = 256
BEV_W = 256
BATCH = 4
NROWS = BATCH * BEV_H * BEV_W

NC = 2
NS = 16
NW = NC * NS

N_PAD = 114688
P_TILE = N_PAD // NS
BLK = 512
NBLK = P_TILE // BLK

CHUNK = 32768
NCHUNKS = 8
NCPC = NCHUNKS // NC
ROWS_PER_TILE = CHUNK // NS

MM_BLK = 1024
MM_GRID = N_PAD // MM_BLK


def _matmul_tc(featp, kidxp, x0p, x1p, bp, kern2):
    def body(feat_ref, kidx_ref, x0_ref, x1_ref, b_ref, kern_ref,
             sf_ref, flat_ref):
        feat = feat_ref[...]
        kidx = kidx_ref[...]
        parts = [
            jnp.where(kidx == k, feat, 0.0).astype(jnp.bfloat16)
            for k in range(NK)
        ]
        big = jnp.concatenate(parts, axis=1)
        kern = kern_ref[...].astype(jnp.bfloat16)
        sf_ref[...] = jnp.dot(
            big, kern, preferred_element_type=jnp.float32
        ).astype(jnp.bfloat16)
        flat_ref[...] = (b_ref[...] * (BEV_H * BEV_W)
                         + x0_ref[...] * BEV_W + x1_ref[...])

    int_spec = pl.BlockSpec((MM_BLK, 1), lambda i: (i, 0))
    sf, flat2 = pl.pallas_call(
        body,
        grid=(MM_GRID,),
        in_specs=[
            pl.BlockSpec((MM_BLK, CIN), lambda i: (i, 0)),
            int_spec, int_spec, int_spec, int_spec,
            pl.BlockSpec((NK * CIN, COUT), lambda i: (0, 0)),
        ],
        out_specs=[
            pl.BlockSpec((MM_BLK, COUT), lambda i: (i, 0)),
            pl.BlockSpec((MM_BLK, 1), lambda i: (i, 0)),
        ],
        out_shape=[
            jax.ShapeDtypeStruct((N_PAD, COUT), jnp.bfloat16),
            jax.ShapeDtypeStruct((N_PAD, 1), jnp.int32),
        ],
    )(featp, kidxp, x0p, x1p, bp, kern2)
    return sf, flat2.reshape(N_PAD)


def _scatter_sc(rows, flat, zeros_hbm):
    mesh = plsc.VectorSubcoreMesh(
        core_axis_name="c", subcore_axis_name="s",
        num_cores=NC, num_subcores=NS)

    @functools.partial(
        pl.kernel,
        out_type=jax.ShapeDtypeStruct((NCHUNKS * CHUNK, COUT), jnp.bfloat16),
        mesh=mesh,
        compiler_params=pltpu.CompilerParams(use_tc_tiling_on_sc=False),
        scratch_types=[
            pltpu.VMEM((P_TILE,), jnp.int32),
            pltpu.VMEM((BLK, COUT), jnp.bfloat16),
            pltpu.VMEM((BLK,), jnp.int32),
            pltpu.VMEM_SHARED((CHUNK + 8, COUT), jnp.bfloat16),
        ],
    )
    def k(rows_hbm, flat_hbm, zsrc_hbm, bev_hbm, idx_v, rows_v, lidx, acc):
        cid = lax.axis_index("c")
        sid = lax.axis_index("s")
        base_pt = sid * P_TILE

        pltpu.sync_copy(flat_hbm.at[pl.ds(base_pt, P_TILE)], idx_v)

        def per_chunk(p, carry):
            chunk_id = cid * NCPC + p
            chunk_base = chunk_id * CHUNK

            pltpu.sync_copy(
                zsrc_hbm,
                acc.at[pl.ds(sid * ROWS_PER_TILE, ROWS_PER_TILE), :])
            plsc.subcore_barrier()

            for bkt in range(NBLK):
                pltpu.sync_copy(
                    rows_hbm.at[pl.ds(base_pt + bkt * BLK, BLK), :], rows_v)

                def sub_body(i2, c, _bkt=bkt):
                    off = _bkt * BLK + i2 * 16
                    v = idx_v[pl.ds(off, 16)]
                    loc = v - chunk_base
                    ok = (loc >= 0) & (loc < CHUNK)
                    loc = jnp.where(ok, loc, CHUNK)
                    lidx[pl.ds(i2 * 16, 16)] = loc
                    return c
                lax.fori_loop(0, BLK // 16, sub_body, 0)
                pltpu.sync_copy(rows_v, acc.at[lidx], add=True)
            plsc.subcore_barrier()

            pltpu.sync_copy(
                acc.at[pl.ds(sid * ROWS_PER_TILE, ROWS_PER_TILE), :],
                bev_hbm.at[pl.ds(chunk_base + sid * ROWS_PER_TILE,
                                 ROWS_PER_TILE), :])
            plsc.subcore_barrier()
            return carry

        lax.fori_loop(0, NCPC, per_chunk, 0)

    return k(rows, flat, zeros_hbm)


def _transpose_tc(bev):
    HW = BEV_H * BEV_W
    TB = 4096

    def body(in_ref, out_ref):
        out_ref[...] = in_ref[...].astype(jnp.float32).T[None]

    return pl.pallas_call(
        body,
        grid=(BATCH, HW // TB),
        in_specs=[pl.BlockSpec((TB, COUT),
                               lambda b, i: (b * (HW // TB) + i, 0))],
        out_specs=pl.BlockSpec((1, COUT, TB), lambda b, i: (b, 0, i)),
        out_shape=jax.ShapeDtypeStruct((BATCH, COUT, HW), jnp.float32),
    )(bev)


def kernel(features, coords_d0, coords_d1, coords_d2, coords_b, stride,
           kernel):
    kidx = (coords_d1 // stride).astype(jnp.int32)
    x0 = (coords_d0 // stride).astype(jnp.int32)
    x1 = (coords_d2 // stride).astype(jnp.int32)
    b = coords_b.astype(jnp.int32)

    pad = N_PAD - N_POINTS
    featp = jnp.concatenate([features, jnp.zeros((pad, CIN), jnp.float32)])
    zi = jnp.zeros((pad,), jnp.int32)
    kidxp = jnp.concatenate([kidx, zi]).reshape(N_PAD, 1)
    x0p = jnp.concatenate([x0, zi]).reshape(N_PAD, 1)
    x1p = jnp.concatenate([x1, zi]).reshape(N_PAD, 1)
    bp = jnp.concatenate([b, zi - 1]).reshape(N_PAD, 1)
    kern2 = kernel.reshape(NK * CIN, COUT)

    sf, flat = _matmul_tc(featp, kidxp, x0p, x1p, bp, kern2)

    zeros_hbm = jnp.zeros((ROWS_PER_TILE, COUT), jnp.bfloat16)
    bev = _scatter_sc(sf, flat, zeros_hbm)

    out = _transpose_tc(bev)
    return out.reshape(BATCH, COUT, BEV_H, BEV_W)


# speedup vs baseline: 2.5584x; 1.1152x over previous
import functools

import jax
import jax.numpy as jnp
from jax import lax
from jax.experimental import pallas as pl
from jax.experimental.pallas import tpu as pltpu
from jax.experimental.pallas import tpu_sc as plsc

N_POINTS = 100000
CIN = 64
COUT = 64
NK = 16
BEV_H = 256
BEV_W = 256
BATCH = 4
NROWS = BATCH * BEV_H * BEV_W

NC = 2
NS = 16

N_PAD = 114688
P_TILE = N_PAD // NS
BLK = 512
NBLK = P_TILE // BLK

CHUNK = 32768
NCHUNKS = 8
NCPC = NCHUNKS // NC
ROWS_PER_TILE = CHUNK // NS

MM_BLK = 1024
MM_GRID = N_PAD // MM_BLK
MM_LAST = (N_POINTS - 1) // MM_BLK


def _matmul_tc(features, kidx, x0, x1, b, kern2):
    def body(feat_ref, kidx_ref, x0_ref, x1_ref, b_ref, kern_ref,
             sf_ref, flat_ref):
        i = pl.program_id(0)
        feat = feat_ref[...].astype(jnp.bfloat16)
        kidx = kidx_ref[...]
        ks = lax.broadcasted_iota(jnp.int32, (1, NK), 1)
        oh = (kidx == ks).astype(jnp.bfloat16)
        r1 = lax.broadcasted_iota(jnp.int32, (NK, NK * CIN), 0)
        r2 = lax.broadcasted_iota(jnp.int32, (NK, NK * CIN), 1) // CIN
        expander = (r1 == r2).astype(jnp.bfloat16)
        ohbig = jnp.dot(oh, expander,
                        preferred_element_type=jnp.float32
                        ).astype(jnp.bfloat16)
        big = jnp.concatenate([feat] * NK, axis=1) * ohbig
        kern = kern_ref[...].astype(jnp.bfloat16)
        sf_ref[...] = jnp.dot(big, kern,
                              preferred_element_type=jnp.float32
                              ).astype(jnp.bfloat16)
        rowid = i * MM_BLK + lax.broadcasted_iota(jnp.int32, (MM_BLK, 1), 0)
        flat = (b_ref[...] * (BEV_H * BEV_W)
                + x0_ref[...] * BEV_W + x1_ref[...])
        flat_ref[...] = jnp.where(rowid < N_POINTS, flat, -1)

    def pt_map(i):
        return (jnp.minimum(i, MM_LAST), 0)

    int_spec = pl.BlockSpec((MM_BLK, 1), pt_map)
    sf, flat2 = pl.pallas_call(
        body,
        grid=(MM_GRID,),
        in_specs=[
            pl.BlockSpec((MM_BLK, CIN), pt_map),
            int_spec, int_spec, int_spec, int_spec,
            pl.BlockSpec((NK * CIN, COUT), lambda i: (0, 0)),
        ],
        out_specs=[
            pl.BlockSpec((MM_BLK, COUT), lambda i: (i, 0)),
            pl.BlockSpec((MM_BLK, 1), lambda i: (i, 0)),
        ],
        out_shape=[
            jax.ShapeDtypeStruct((N_PAD, COUT), jnp.bfloat16),
            jax.ShapeDtypeStruct((N_PAD, 1), jnp.int32),
        ],
    )(features, kidx, x0, x1, b, kern2)
    return sf, flat2.reshape(N_PAD)


def _scatter_sc(rows, flat, zeros_hbm):
    mesh = plsc.VectorSubcoreMesh(
        core_axis_name="c", subcore_axis_name="s",
        num_cores=NC, num_subcores=NS)

    @functools.partial(
        pl.kernel,
        out_type=jax.ShapeDtypeStruct((NCHUNKS * CHUNK, COUT), jnp.bfloat16),
        mesh=mesh,
        compiler_params=pltpu.CompilerParams(use_tc_tiling_on_sc=False),
        scratch_types=[
            pltpu.VMEM((P_TILE,), jnp.int32),
            pltpu.VMEM((BLK, COUT), jnp.bfloat16),
            pltpu.VMEM((BLK, COUT), jnp.bfloat16),
            pltpu.VMEM((BLK,), jnp.int32),
            pltpu.VMEM((BLK,), jnp.int32),
            pltpu.VMEM_SHARED((CHUNK + 8, COUT), jnp.bfloat16),
            pltpu.SemaphoreType.DMA,
            pltpu.SemaphoreType.DMA,
            pltpu.SemaphoreType.DMA,
            pltpu.SemaphoreType.DMA,
        ],
    )
    def k(rows_hbm, flat_hbm, zsrc_hbm, bev_hbm, idx_v, rows0, rows1,
          lidx0, lidx1, acc, lsem0, lsem1, ssem0, ssem1):
        cid = lax.axis_index("c")
        sid = lax.axis_index("s")
        base_pt = sid * P_TILE
        rows_b = (rows0, rows1)
        lidx_b = (lidx0, lidx1)
        lsem_b = (lsem0, lsem1)
        ssem_b = (ssem0, ssem1)

        pltpu.sync_copy(flat_hbm.at[pl.ds(base_pt, P_TILE)], idx_v)

        def per_chunk(p, carry):
            chunk_id = cid * NCPC + p
            chunk_base = chunk_id * CHUNK

            pltpu.sync_copy(
                zsrc_hbm,
                acc.at[pl.ds(sid * ROWS_PER_TILE, ROWS_PER_TILE), :])
            plsc.subcore_barrier()

            loads = [None, None]
            scats = [None, None]
            loads[0] = pltpu.async_copy(
                rows_hbm.at[pl.ds(base_pt, BLK), :], rows0, lsem0)
            for bkt in range(NBLK):
                bb = bkt & 1
                nb = 1 - bb
                if bkt + 1 < NBLK:
                    if scats[nb] is not None:
                        scats[nb].wait()
                        scats[nb] = None
                    loads[nb] = pltpu.async_copy(
                        rows_hbm.at[pl.ds(base_pt + (bkt + 1) * BLK, BLK), :],
                        rows_b[nb], lsem_b[nb])
                loads[bb].wait()

                def sub_body(i2, c, _bkt=bkt, _bb=bb):
                    off = _bkt * BLK + i2 * 16
                    v = idx_v[pl.ds(off, 16)]
                    loc = v - chunk_base
                    ok = (loc >= 0) & (loc < CHUNK)
                    loc = jnp.where(ok, loc, CHUNK)
                    lidx_b[_bb][pl.ds(i2 * 16, 16)] = loc
                    return c
                lax.fori_loop(0, BLK // 16, sub_body, 0)
                if scats[bb] is not None:
                    scats[bb].wait()
                scats[bb] = pltpu.async_copy(
                    rows_b[bb], acc.at[lidx_b[bb]], ssem_b[bb], add=True)
            for s in scats:
                if s is not None:
                    s.wait()
            plsc.subcore_barrier()

            pltpu.sync_copy(
                acc.at[pl.ds(sid * ROWS_PER_TILE, ROWS_PER_TILE), :],
                bev_hbm.at[pl.ds(chunk_base + sid * ROWS_PER_TILE,
                                 ROWS_PER_TILE), :])
            plsc.subcore_barrier()
            return carry

        lax.fori_loop(0, NCPC, per_chunk, 0)

    return k(rows, flat, zeros_hbm)


def _transpose_tc(bev):
    HW = BEV_H * BEV_W
    TB = 4096
    SH = TB // BEV_W

    def body(in_ref, out_ref):
        x = in_ref[...].astype(jnp.float32).T
        out_ref[...] = x.reshape(1, COUT, SH, BEV_W)

    return pl.pallas_call(
        body,
        grid=(BATCH, HW // TB),
        in_specs=[pl.BlockSpec((TB, COUT),
                               lambda b, i: (b * (HW // TB) + i, 0))],
        out_specs=pl.BlockSpec((1, COUT, SH, BEV_W),
                               lambda b, i: (b, 0, i, 0)),
        out_shape=jax.ShapeDtypeStruct((BATCH, COUT, BEV_H, BEV_W),
                                       jnp.float32),
    )(bev)


def kernel(features, coords_d0, coords_d1, coords_d2, coords_b, stride,
           kernel):
    kidx = (coords_d1 // stride).astype(jnp.int32).reshape(N_POINTS, 1)
    x0 = (coords_d0 // stride).astype(jnp.int32).reshape(N_POINTS, 1)
    x1 = (coords_d2 // stride).astype(jnp.int32).reshape(N_POINTS, 1)
    b = coords_b.astype(jnp.int32).reshape(N_POINTS, 1)
    kern2 = kernel.reshape(NK * CIN, COUT)

    sf, flat = _matmul_tc(features, kidx, x0, x1, b, kern2)

    zeros_hbm = jnp.zeros((ROWS_PER_TILE, COUT), jnp.bfloat16)
    bev = _scatter_sc(sf, flat, zeros_hbm)

    return _transpose_tc(bev)


# speedup vs baseline: 3.2177x; 1.2577x over previous
import functools

import jax
import jax.numpy as jnp
from jax import lax
from jax.experimental import pallas as pl
from jax.experimental.pallas import tpu as pltpu
from jax.experimental.pallas import tpu_sc as plsc

N_POINTS = 100000
CIN = 64
COUT = 64
NK = 16
BEV_H = 256
BEV_W = 256
BATCH = 4
NROWS = BATCH * BEV_H * BEV_W

NC = 2
NS = 16

N_PAD = 114688
P_TILE = N_PAD // NS
BLK = 512
NBLK = P_TILE // BLK

CHUNK = 32768
NCHUNKS = 8
NCPC = NCHUNKS // NC
ROWS_PER_TILE = CHUNK // NS

MM_BLK = 1024
MM_GRID = N_PAD // MM_BLK
MM_LAST = (N_POINTS - 1) // MM_BLK


def _matmul_tc(features, kidx, x0, x1, b, kern2):
    def body(feat_ref, kidx_ref, x0_ref, x1_ref, b_ref, kern_ref,
             sf_ref, flat_ref):
        i = pl.program_id(0)
        feat = feat_ref[...].astype(jnp.bfloat16)
        kidx = kidx_ref[...].reshape(MM_BLK, 1)
        ks = lax.broadcasted_iota(jnp.int32, (1, NK), 1)
        oh = (kidx == ks).astype(jnp.bfloat16)
        r1 = lax.broadcasted_iota(jnp.int32, (NK, NK * CIN), 0)
        r2 = lax.broadcasted_iota(jnp.int32, (NK, NK * CIN), 1) // CIN
        expander = (r1 == r2).astype(jnp.bfloat16)
        ohbig = jnp.dot(oh, expander,
                        preferred_element_type=jnp.float32
                        ).astype(jnp.bfloat16)
        big = jnp.concatenate([feat] * NK, axis=1) * ohbig
        kern = kern_ref[...].astype(jnp.bfloat16)
        sf_ref[...] = jnp.dot(big, kern,
                              preferred_element_type=jnp.float32
                              ).astype(jnp.bfloat16)
        rowid = i * MM_BLK + lax.broadcasted_iota(jnp.int32, (MM_BLK,), 0)
        flat = (b_ref[...] * (BEV_H * BEV_W)
                + x0_ref[...] * BEV_W + x1_ref[...])
        flat_ref[...] = jnp.where(rowid < N_POINTS, flat, -1)

    def pt_map(i):
        return (jnp.minimum(i, MM_LAST), 0)

    int_spec = pl.BlockSpec((MM_BLK,), lambda i: (jnp.minimum(i, MM_LAST),))
    sf, flat2 = pl.pallas_call(
        body,
        grid=(MM_GRID,),
        in_specs=[
            pl.BlockSpec((MM_BLK, CIN), pt_map),
            int_spec, int_spec, int_spec, int_spec,
            pl.BlockSpec((NK * CIN, COUT), lambda i: (0, 0)),
        ],
        out_specs=[
            pl.BlockSpec((MM_BLK, COUT), lambda i: (i, 0)),
            pl.BlockSpec((MM_BLK,), lambda i: (i,)),
        ],
        out_shape=[
            jax.ShapeDtypeStruct((N_PAD, COUT), jnp.bfloat16),
            jax.ShapeDtypeStruct((N_PAD,), jnp.int32),
        ],
    )(features, kidx, x0, x1, b, kern2)
    return sf, flat2


def _scatter_sc(rows, flat, zeros_hbm):
    mesh = plsc.VectorSubcoreMesh(
        core_axis_name="c", subcore_axis_name="s",
        num_cores=NC, num_subcores=NS)

    @functools.partial(
        pl.kernel,
        out_type=jax.ShapeDtypeStruct((NCHUNKS * CHUNK, COUT), jnp.bfloat16),
        mesh=mesh,
        compiler_params=pltpu.CompilerParams(use_tc_tiling_on_sc=False),
        scratch_types=[
            pltpu.VMEM((P_TILE,), jnp.int32),
            pltpu.VMEM((BLK, COUT), jnp.bfloat16),
            pltpu.VMEM((BLK, COUT), jnp.bfloat16),
            pltpu.VMEM((BLK,), jnp.int32),
            pltpu.VMEM((BLK,), jnp.int32),
            pltpu.VMEM_SHARED((CHUNK + 8, COUT), jnp.bfloat16),
            pltpu.SemaphoreType.DMA,
            pltpu.SemaphoreType.DMA,
            pltpu.SemaphoreType.DMA,
            pltpu.SemaphoreType.DMA,
        ],
    )
    def k(rows_hbm, flat_hbm, zsrc_hbm, bev_hbm, idx_v, rows0, rows1,
          lidx0, lidx1, acc, lsem0, lsem1, ssem0, ssem1):
        cid = lax.axis_index("c")
        sid = lax.axis_index("s")
        base_pt = sid * P_TILE
        rows_b = (rows0, rows1)
        lidx_b = (lidx0, lidx1)
        lsem_b = (lsem0, lsem1)
        ssem_b = (ssem0, ssem1)

        pltpu.sync_copy(flat_hbm.at[pl.ds(base_pt, P_TILE)], idx_v)

        def per_chunk(p, carry):
            chunk_id = cid * NCPC + p
            chunk_base = chunk_id * CHUNK

            pltpu.sync_copy(
                zsrc_hbm,
                acc.at[pl.ds(sid * ROWS_PER_TILE, ROWS_PER_TILE), :])
            plsc.subcore_barrier()

            loads = [None, None]
            scats = [None, None]
            loads[0] = pltpu.async_copy(
                rows_hbm.at[pl.ds(base_pt, BLK), :], rows0, lsem0)
            for bkt in range(NBLK):
                bb = bkt & 1
                nb = 1 - bb
                if bkt + 1 < NBLK:
                    if scats[nb] is not None:
                        scats[nb].wait()
                        scats[nb] = None
                    loads[nb] = pltpu.async_copy(
                        rows_hbm.at[pl.ds(base_pt + (bkt + 1) * BLK, BLK), :],
                        rows_b[nb], lsem_b[nb])
                loads[bb].wait()

                def sub_body(i2, c, _bkt=bkt, _bb=bb):
                    off = _bkt * BLK + i2 * 16
                    v = idx_v[pl.ds(off, 16)]
                    loc = v - chunk_base
                    ok = (loc >= 0) & (loc < CHUNK)
                    loc = jnp.where(ok, loc, CHUNK)
                    lidx_b[_bb][pl.ds(i2 * 16, 16)] = loc
                    return c
                lax.fori_loop(0, BLK // 16, sub_body, 0)
                if scats[bb] is not None:
                    scats[bb].wait()
                scats[bb] = pltpu.async_copy(
                    rows_b[bb], acc.at[lidx_b[bb]], ssem_b[bb], add=True)
            for s in scats:
                if s is not None:
                    s.wait()
            plsc.subcore_barrier()

            pltpu.sync_copy(
                acc.at[pl.ds(sid * ROWS_PER_TILE, ROWS_PER_TILE), :],
                bev_hbm.at[pl.ds(chunk_base + sid * ROWS_PER_TILE,
                                 ROWS_PER_TILE), :])
            plsc.subcore_barrier()
            return carry

        lax.fori_loop(0, NCPC, per_chunk, 0)

    return k(rows, flat, zeros_hbm)


def _transpose_tc(bev):
    HW = BEV_H * BEV_W
    TB = 4096
    SH = TB // BEV_W

    def body(in_ref, out_ref):
        x = in_ref[...].astype(jnp.float32).T
        out_ref[...] = x.reshape(1, COUT, SH, BEV_W)

    return pl.pallas_call(
        body,
        grid=(BATCH, HW // TB),
        in_specs=[pl.BlockSpec((TB, COUT),
                               lambda b, i: (b * (HW // TB) + i, 0))],
        out_specs=pl.BlockSpec((1, COUT, SH, BEV_W),
                               lambda b, i: (b, 0, i, 0)),
        out_shape=jax.ShapeDtypeStruct((BATCH, COUT, BEV_H, BEV_W),
                                       jnp.float32),
    )(bev)


def kernel(features, coords_d0, coords_d1, coords_d2, coords_b, stride,
           kernel):
    kidx = (coords_d1 // stride).astype(jnp.int32)
    x0 = (coords_d0 // stride).astype(jnp.int32)
    x1 = (coords_d2 // stride).astype(jnp.int32)
    b = coords_b.astype(jnp.int32)
    kern2 = kernel.reshape(NK * CIN, COUT)

    sf, flat = _matmul_tc(features, kidx, x0, x1, b, kern2)

    zeros_hbm = jnp.zeros((ROWS_PER_TILE, COUT), jnp.bfloat16)
    bev = _scatter_sc(sf, flat, zeros_hbm)

    return _transpose_tc(bev)


# speedup vs baseline: 3.2190x; 1.0004x over previous
import functools

import jax
import jax.numpy as jnp
from jax import lax
from jax.experimental import pallas as pl
from jax.experimental.pallas import tpu as pltpu
from jax.experimental.pallas import tpu_sc as plsc

N_POINTS = 100000
CIN = 64
COUT = 64
NK = 16
BEV_H = 256
BEV_W = 256
BATCH = 4
NROWS = BATCH * BEV_H * BEV_W

NC = 2
NS = 16

N_PAD = 114688
P_TILE = N_PAD // NS
BLK = 512
NBLK = P_TILE // BLK

CHUNK = 32768
NCHUNKS = 8
NCPC = NCHUNKS // NC
ROWS_PER_TILE = CHUNK // NS

MM_BLK = 1024
MM_GRID = N_PAD // MM_BLK
MM_LAST = (N_POINTS - 1) // MM_BLK


def _matmul_tc(features, kidx, x0, x1, b, kern2):
    def body(feat_ref, kidx_ref, x0_ref, x1_ref, b_ref, kern_ref,
             sf_ref, flat_ref):
        i = pl.program_id(0)
        feat = feat_ref[...].astype(jnp.bfloat16)
        kidx = kidx_ref[...].reshape(MM_BLK, 1)
        ks = lax.broadcasted_iota(jnp.int32, (1, NK), 1)
        oh = (kidx == ks).astype(jnp.bfloat16)
        r1 = lax.broadcasted_iota(jnp.int32, (NK, NK * CIN), 0)
        r2 = lax.broadcasted_iota(jnp.int32, (NK, NK * CIN), 1) // CIN
        expander = (r1 == r2).astype(jnp.bfloat16)
        ohbig = jnp.dot(oh, expander,
                        preferred_element_type=jnp.float32
                        ).astype(jnp.bfloat16)
        big = jnp.concatenate([feat] * NK, axis=1) * ohbig
        sf_ref[...] = jnp.dot(big, kern_ref[...],
                              preferred_element_type=jnp.float32
                              ).astype(jnp.bfloat16)
        rowid = i * MM_BLK + lax.broadcasted_iota(jnp.int32, (MM_BLK,), 0)
        flat = (b_ref[...] * (BEV_H * BEV_W)
                + x0_ref[...] * BEV_W + x1_ref[...])
        flat_ref[...] = jnp.where(rowid < N_POINTS, flat, -1)

    def pt_map(i):
        return (jnp.minimum(i, MM_LAST), 0)

    int_spec = pl.BlockSpec((MM_BLK,), lambda i: (jnp.minimum(i, MM_LAST),))
    sf, flat2 = pl.pallas_call(
        body,
        grid=(MM_GRID,),
        in_specs=[
            pl.BlockSpec((MM_BLK, CIN), pt_map),
            int_spec, int_spec, int_spec, int_spec,
            pl.BlockSpec((NK * CIN, COUT), lambda i: (0, 0)),
        ],
        out_specs=[
            pl.BlockSpec((MM_BLK, COUT), lambda i: (i, 0)),
            pl.BlockSpec((MM_BLK,), lambda i: (i,)),
        ],
        out_shape=[
            jax.ShapeDtypeStruct((N_PAD, COUT), jnp.bfloat16),
            jax.ShapeDtypeStruct((N_PAD,), jnp.int32),
        ],
    )(features, kidx, x0, x1, b, kern2)
    return sf, flat2


def _scatter_sc(rows, flat, zeros_hbm):
    mesh = plsc.VectorSubcoreMesh(
        core_axis_name="c", subcore_axis_name="s",
        num_cores=NC, num_subcores=NS)

    @functools.partial(
        pl.kernel,
        out_type=jax.ShapeDtypeStruct((NCHUNKS * CHUNK, COUT), jnp.bfloat16),
        mesh=mesh,
        compiler_params=pltpu.CompilerParams(use_tc_tiling_on_sc=False),
        scratch_types=[
            pltpu.VMEM((P_TILE,), jnp.int32),
            pltpu.VMEM((BLK, COUT), jnp.bfloat16),
            pltpu.VMEM((BLK, COUT), jnp.bfloat16),
            pltpu.VMEM((BLK,), jnp.int32),
            pltpu.VMEM((BLK,), jnp.int32),
            pltpu.VMEM_SHARED((CHUNK + 8, COUT), jnp.bfloat16),
            pltpu.SemaphoreType.DMA,
            pltpu.SemaphoreType.DMA,
            pltpu.SemaphoreType.DMA,
            pltpu.SemaphoreType.DMA,
        ],
    )
    def k(rows_hbm, flat_hbm, zsrc_hbm, bev_hbm, idx_v, rows0, rows1,
          lidx0, lidx1, acc, lsem0, lsem1, ssem0, ssem1):
        cid = lax.axis_index("c")
        sid = lax.axis_index("s")
        base_pt = sid * P_TILE
        rows_b = (rows0, rows1)
        lidx_b = (lidx0, lidx1)
        lsem_b = (lsem0, lsem1)
        ssem_b = (ssem0, ssem1)

        pltpu.sync_copy(flat_hbm.at[pl.ds(base_pt, P_TILE)], idx_v)

        def per_chunk(p, carry):
            chunk_id = cid * NCPC + p
            chunk_base = chunk_id * CHUNK

            pltpu.sync_copy(
                zsrc_hbm,
                acc.at[pl.ds(sid * ROWS_PER_TILE, ROWS_PER_TILE), :])
            plsc.subcore_barrier()

            loads = [None, None]
            scats = [None, None]
            loads[0] = pltpu.async_copy(
                rows_hbm.at[pl.ds(base_pt, BLK), :], rows0, lsem0)
            for bkt in range(NBLK):
                bb = bkt & 1
                nb = 1 - bb
                if bkt + 1 < NBLK:
                    if scats[nb] is not None:
                        scats[nb].wait()
                        scats[nb] = None
                    loads[nb] = pltpu.async_copy(
                        rows_hbm.at[pl.ds(base_pt + (bkt + 1) * BLK, BLK), :],
                        rows_b[nb], lsem_b[nb])
                loads[bb].wait()

                def sub_body(i2, c, _bkt=bkt, _bb=bb):
                    off = _bkt * BLK + i2 * 16
                    v = idx_v[pl.ds(off, 16)]
                    loc = v - chunk_base
                    ok = (loc >= 0) & (loc < CHUNK)
                    loc = jnp.where(ok, loc, CHUNK)
                    lidx_b[_bb][pl.ds(i2 * 16, 16)] = loc
                    return c
                lax.fori_loop(0, BLK // 16, sub_body, 0)
                if scats[bb] is not None:
                    scats[bb].wait()
                scats[bb] = pltpu.async_copy(
                    rows_b[bb], acc.at[lidx_b[bb]], ssem_b[bb], add=True)
            for s in scats:
                if s is not None:
                    s.wait()
            plsc.subcore_barrier()

            pltpu.sync_copy(
                acc.at[pl.ds(sid * ROWS_PER_TILE, ROWS_PER_TILE), :],
                bev_hbm.at[pl.ds(chunk_base + sid * ROWS_PER_TILE,
                                 ROWS_PER_TILE), :])
            plsc.subcore_barrier()
            return carry

        lax.fori_loop(0, NCPC, per_chunk, 0)

    return k(rows, flat, zeros_hbm)


def _transpose_tc(bev):
    HW = BEV_H * BEV_W
    TB = 4096
    SH = TB // BEV_W

    def body(in_ref, out_ref):
        x = in_ref[...].astype(jnp.float32).T
        out_ref[...] = x.reshape(1, COUT, SH, BEV_W)

    return pl.pallas_call(
        body,
        grid=(BATCH, HW // TB),
        in_specs=[pl.BlockSpec((TB, COUT),
                               lambda b, i: (b * (HW // TB) + i, 0))],
        out_specs=pl.BlockSpec((1, COUT, SH, BEV_W),
                               lambda b, i: (b, 0, i, 0)),
        out_shape=jax.ShapeDtypeStruct((BATCH, COUT, BEV_H, BEV_W),
                                       jnp.float32),
    )(bev)


def kernel(features, coords_d0, coords_d1, coords_d2, coords_b, stride,
           kernel):
    kidx = (coords_d1 // stride).astype(jnp.int32)
    x0 = (coords_d0 // stride).astype(jnp.int32)
    x1 = (coords_d2 // stride).astype(jnp.int32)
    b = coords_b.astype(jnp.int32)
    kern2 = kernel.reshape(NK * CIN, COUT).astype(jnp.bfloat16)

    sf, flat = _matmul_tc(features, kidx, x0, x1, b, kern2)

    zeros_hbm = jnp.zeros((ROWS_PER_TILE, COUT), jnp.bfloat16)
    bev = _scatter_sc(sf, flat, zeros_hbm)

    return _transpose_tc(bev)
